# baseline (speedup 1.0000x reference)
import functools

import numpy as np

import jax
import jax.numpy as jnp
from jax import lax
from jax.experimental import pallas as pl
from jax.experimental.pallas import tpu as pltpu
from jax.experimental.pallas import tpu_sc as plsc

T = 3
N = 100000
D = 128
B = 10000
S1, S2 = 5, 2
H1, H2 = 512, 10
OUT = 32
VTH = 1.0

NC, NS = 2, 16
NW = NC * NS
ROWS_T = B * (1 + S1 + S1 * S2)
CHUNK = 128
KCH = 40
RPW = CHUNK * KCH
GT = RPW * NW

BLK = 1000
NB = B // BLK

BH = B // 2
KCHH = 20
RPWH = CHUNK * KCHH
GH = RPWH * NW
NBH = BH // BLK


def _build_oidx(nb: int, kch: int) -> np.ndarray:
    gt = NW * kch * CHUNK
    p = np.arange(gt, dtype=np.int64)
    out = p.copy()
    s1lo, s1hi = nb, nb + nb * S1
    q = p - s1lo
    sel = (p >= s1lo) & (p < s1hi)
    out[sel] = s1lo + (q[sel] % S1) * nb + q[sel] // S1
    s2lo, s2hi = s1hi, s1hi + nb * S1 * S2
    q = p - s2lo
    sel = (p >= s2lo) & (p < s2hi)
    out[sel] = s2lo + (q[sel] % (S1 * S2)) * nb + q[sel] // (S1 * S2)
    return out.reshape(NW, kch, CHUNK).astype(np.int32)


_OIDX = _build_oidx(B, KCH)
_OIDXH = _build_oidx(BH, KCHH)
_PADIDX = np.arange(GT - ROWS_T, dtype=np.int32)
_PADIDXH = np.arange(GH - BH * 16, dtype=np.int32)


def _sc_gather_build(kch: int):
    rpw = CHUNK * kch
    gt = rpw * NW
    mesh = plsc.VectorSubcoreMesh(core_axis_name="c", subcore_axis_name="s")

    @functools.partial(
        pl.kernel,
        out_type=jax.ShapeDtypeStruct((gt, D), jnp.float32),
        mesh=mesh,
        scratch_types=[
            pltpu.VMEM((rpw,), jnp.int32),
            pltpu.VMEM((kch, CHUNK), jnp.int32),
            pltpu.VMEM((4, CHUNK, D), jnp.float32),
            pltpu.SemaphoreType.DMA,
            pltpu.SemaphoreType.DMA,
            pltpu.SemaphoreType.DMA,
            pltpu.SemaphoreType.DMA,
            pltpu.SemaphoreType.DMA,
            pltpu.SemaphoreType.DMA,
            pltpu.SemaphoreType.DMA,
            pltpu.SemaphoreType.DMA,
        ],
    )
    def sc_gather(table, idx, oidx, out, iv, ov, rv,
                  g0, g1, g2, g3, w0, w1, w2, w3):
        wid = lax.axis_index("s") * NC + lax.axis_index("c")
        pltpu.sync_copy(idx.at[pl.ds(wid * rpw, rpw)], iv)
        pltpu.sync_copy(oidx.at[wid], ov)
        gsem = (g0, g1, g2, g3)
        wsem = (w0, w1, w2, w3)

        def gstart(c, s):
            pltpu.make_async_copy(
                table.at[iv.at[pl.ds(c * CHUNK, CHUNK)]],
                rv.at[s], gsem[s]).start()

        def gwait(s):
            pltpu.make_async_copy(
                table.at[iv.at[pl.ds(0, CHUNK)]], rv.at[s], gsem[s]).wait()

        def wstart(c, s):
            pltpu.make_async_copy(rv.at[s], out.at[ov.at[c]],
                                  wsem[s]).start()

        def wwait(s):
            pltpu.make_async_copy(rv.at[0], out.at[ov.at[0]],
                                  wsem[s]).wait()

        gstart(0, 0)
        gstart(1, 1)
        for s in range(4):
            c = s
            gwait(s)
            wstart(c, s)
            if c >= 2:
                wwait((c + 2) % 4)
            gstart(c + 2, (s + 2) % 4)

        def body(j, carry):
            for s in range(4):
                c = 4 * j + s
                gwait(s)
                wstart(c, s)
                wwait((s + 2) % 4)
                gstart(c + 2, (s + 2) % 4)
            return carry

        lax.fori_loop(1, kch // 4 - 1, body, 0)
        for s in range(4):
            c = kch - 4 + s
            gwait(s)
            wstart(c, s)
            wwait((s + 2) % 4)
            if c + 2 < kch:
                gstart(c + 2, (s + 2) % 4)
        wwait(2)
        wwait(3)

    return sc_gather


_sc_gather = _sc_gather_build(KCH)
_sc_gather_h = _sc_gather_build(KCHH)


def _tc_body_build(t: int):
    def body(h0, h10, h11, h12, h13, h14,
             h20, h21, h22, h23, h24, h25, h26, h27, h28, h29,
             w1, w2, wp, prev, out):
        h1b = (h10, h11, h12, h13, h14)
        h2b = (h20, h21, h22, h23, h24, h25, h26, h27, h28, h29)
        w1v = w1[...]
        n0 = h1b[0][...]
        for k in range(1, S1):
            n0 = n0 + h1b[k][...]
        a0 = h0[...] + n0 * (1.0 / S1)
        g = (jnp.dot(a0, w1v, preferred_element_type=jnp.float32)
             >= VTH).astype(jnp.float32)
        gs = jnp.zeros((BLK, H1), jnp.float32)
        for k in range(S1):
            a1 = h1b[k][...] + 0.5 * (h2b[2 * k][...] + h2b[2 * k + 1][...])
            s1 = (jnp.dot(a1, w1v, preferred_element_type=jnp.float32)
                  >= VTH).astype(jnp.float32)
            gs = gs + s1
        g2 = g + gs * (1.0 / S1)
        o2 = jnp.dot(g2, w2[...], preferred_element_type=jnp.float32)
        s2 = (o2 >= VTH).astype(jnp.float32)
        out[...] = prev[...] + jnp.dot(s2, wp[0],
                                       preferred_element_type=jnp.float32)

    return body


def _tc_net_build(t: int, nb: int = NB, prev_off: int = 0):
    prev_spec = (pl.BlockSpec((1, OUT), lambda i: (0, 0)) if t == 0
                 else pl.BlockSpec((BLK, OUT),
                                   lambda i, o=prev_off: (o + i, 0)))
    in_specs = (
        [pl.BlockSpec((BLK, D), lambda i: (i, 0))]
        + [pl.BlockSpec((BLK, D), lambda i, k=k: ((1 + k) * nb + i, 0))
           for k in range(S1)]
        + [pl.BlockSpec((BLK, D),
                        lambda i, m=m: ((1 + S1 + m) * nb + i, 0))
           for m in range(S1 * S2)]
        + [
            pl.BlockSpec((D, H1), lambda i: (0, 0)),
            pl.BlockSpec((H1, H2), lambda i: (0, 0)),
            pl.BlockSpec((1, H2, OUT), lambda i, t=t: (t, 0, 0)),
            prev_spec,
        ]
    )
    return pl.pallas_call(
        _tc_body_build(t),
        grid=(nb,),
        in_specs=in_specs,
        out_specs=pl.BlockSpec((BLK, OUT), lambda i: (i, 0)),
        out_shape=jax.ShapeDtypeStruct((nb * BLK, OUT), jnp.float32),
    )


_tc_net_t = [_tc_net_build(t) for t in range(T - 1)]
_tc_net_h = [_tc_net_build(T - 1, nb=NBH, prev_off=h * NBH)
             for h in range(2)]


def kernel(x, nodes, nbr1, nbr2, W1, W2, Wp, bp):
    table = x.reshape(T * N, D)
    nodes_i = nodes.astype(jnp.int32)
    oidx = jnp.asarray(_OIDX)
    pad = jnp.asarray(_PADIDX)
    wp3 = Wp.reshape(T, H2, OUT)

    prev = bp.reshape(1, OUT)
    for t in range(T - 1):
        idx_t = jnp.concatenate(
            [nodes_i, nbr1[t].astype(jnp.int32),
             nbr2[t].astype(jnp.int32), pad]) + (t * N)
        hg = _sc_gather(table, idx_t, oidx)
        args = [hg] * 16 + [W1, W2, wp3, prev]
        prev = _tc_net_t[t](*args)

    t = T - 1
    oidxh = jnp.asarray(_OIDXH)
    padh = jnp.asarray(_PADIDXH)
    n1t = nbr1[t].astype(jnp.int32)
    n2t = nbr2[t].astype(jnp.int32)
    outs = []
    for h in range(2):
        idx_h = jnp.concatenate(
            [nodes_i[h * BH:(h + 1) * BH],
             n1t[h * BH * S1:(h + 1) * BH * S1],
             n2t[h * BH * S1 * S2:(h + 1) * BH * S1 * S2],
             padh]) + (t * N)
        hg = _sc_gather_h(table, idx_h, oidxh)
        args = [hg] * 16 + [W1, W2, wp3, prev]
        outs.append(_tc_net_h[h](*args))
    return jnp.concatenate(outs, axis=0)


# speedup vs baseline: 1.0052x; 1.0052x over previous
import functools

import numpy as np

import jax
import jax.numpy as jnp
from jax import lax
from jax.experimental import pallas as pl
from jax.experimental.pallas import tpu as pltpu
from jax.experimental.pallas import tpu_sc as plsc

T = 3
N = 100000
D = 128
B = 10000
S1, S2 = 5, 2
H1, H2 = 512, 10
OUT = 32
VTH = 1.0

NC, NS = 2, 16
NW = NC * NS
ROWS_T = B * (1 + S1 + S1 * S2)
CHUNK = 128
KCH = 40
RPW = CHUNK * KCH
GT = RPW * NW

BLK = 1000
NB = B // BLK

def _build_oidx() -> np.ndarray:
    p = np.arange(GT, dtype=np.int64)
    out = p.copy()
    s1lo, s1hi = B, B + B * S1
    q = p - s1lo
    sel = (p >= s1lo) & (p < s1hi)
    out[sel] = s1lo + (q[sel] % S1) * B + q[sel] // S1
    s2lo, s2hi = s1hi, s1hi + B * S1 * S2
    q = p - s2lo
    sel = (p >= s2lo) & (p < s2hi)
    out[sel] = s2lo + (q[sel] % (S1 * S2)) * B + q[sel] // (S1 * S2)
    return out.reshape(NW, KCH, CHUNK).astype(np.int32)


_OIDX = _build_oidx()
_PADIDX = np.arange(GT - ROWS_T, dtype=np.int32)


def _sc_gather_build():
    mesh = plsc.VectorSubcoreMesh(core_axis_name="c", subcore_axis_name="s")

    @functools.partial(
        pl.kernel,
        out_type=jax.ShapeDtypeStruct((GT, D), jnp.float32),
        mesh=mesh,
        scratch_types=[
            pltpu.VMEM((RPW,), jnp.int32),
            pltpu.VMEM((KCH, CHUNK), jnp.int32),
            pltpu.VMEM((4, CHUNK, D), jnp.float32),
            pltpu.SemaphoreType.DMA,
            pltpu.SemaphoreType.DMA,
            pltpu.SemaphoreType.DMA,
            pltpu.SemaphoreType.DMA,
            pltpu.SemaphoreType.DMA,
            pltpu.SemaphoreType.DMA,
            pltpu.SemaphoreType.DMA,
            pltpu.SemaphoreType.DMA,
        ],
    )
    def sc_gather(table, idx, oidx, out, iv, ov, rv,
                  g0, g1, g2, g3, w0, w1, w2, w3):
        wid = lax.axis_index("s") * NC + lax.axis_index("c")
        pltpu.sync_copy(idx.at[pl.ds(wid * RPW, RPW)], iv)
        pltpu.sync_copy(oidx.at[wid], ov)
        gsem = (g0, g1, g2, g3)
        wsem = (w0, w1, w2, w3)

        def gstart(c, s):
            pltpu.make_async_copy(
                table.at[iv.at[pl.ds(c * CHUNK, CHUNK)]],
                rv.at[s], gsem[s]).start()

        def gwait(s):
            pltpu.make_async_copy(
                table.at[iv.at[pl.ds(0, CHUNK)]], rv.at[s], gsem[s]).wait()

        def wstart(c, s):
            pltpu.make_async_copy(rv.at[s], out.at[ov.at[c]],
                                  wsem[s]).start()

        def wwait(s):
            pltpu.make_async_copy(rv.at[0], out.at[ov.at[0]],
                                  wsem[s]).wait()

        gstart(0, 0)
        gstart(1, 1)
        for s in range(4):
            c = s
            gwait(s)
            wstart(c, s)
            if c >= 2:
                wwait((c + 2) % 4)
            gstart(c + 2, (s + 2) % 4)

        def body(j, carry):
            for s in range(4):
                c = 4 * j + s
                gwait(s)
                wstart(c, s)
                wwait((s + 2) % 4)
                gstart(c + 2, (s + 2) % 4)
            return carry

        lax.fori_loop(1, KCH // 4 - 1, body, 0)
        for s in range(4):
            c = KCH - 4 + s
            gwait(s)
            wstart(c, s)
            wwait((s + 2) % 4)
            if c + 2 < KCH:
                gstart(c + 2, (s + 2) % 4)
        wwait(2)
        wwait(3)

    return sc_gather


_sc_gather = _sc_gather_build()


def _tc_body_build(t: int):
    def body(h0, h10, h11, h12, h13, h14,
             h20, h21, h22, h23, h24, h25, h26, h27, h28, h29,
             w1, w2, wp, prev, out):
        h1b = (h10, h11, h12, h13, h14)
        h2b = (h20, h21, h22, h23, h24, h25, h26, h27, h28, h29)
        w1v = w1[...]
        n0 = h1b[0][...]
        for k in range(1, S1):
            n0 = n0 + h1b[k][...]
        a0 = h0[...] + n0 * (1.0 / S1)
        g = (jnp.dot(a0, w1v, preferred_element_type=jnp.float32)
             >= VTH).astype(jnp.float32)
        gs = jnp.zeros((BLK, H1), jnp.float32)
        for k in range(S1):
            a1 = h1b[k][...] + 0.5 * (h2b[2 * k][...] + h2b[2 * k + 1][...])
            s1 = (jnp.dot(a1, w1v, preferred_element_type=jnp.float32)
                  >= VTH).astype(jnp.float32)
            gs = gs + s1
        g2 = g + gs * (1.0 / S1)
        o2 = jnp.dot(g2, w2[...], preferred_element_type=jnp.float32)
        s2 = (o2 >= VTH).astype(jnp.float32)
        out[...] = prev[...] + jnp.dot(s2, wp[0],
                                       preferred_element_type=jnp.float32)

    return body


def _tc_net_build(t: int):
    prev_spec = (pl.BlockSpec((1, OUT), lambda i: (0, 0)) if t == 0
                 else pl.BlockSpec((BLK, OUT), lambda i: (i, 0)))
    in_specs = (
        [pl.BlockSpec((BLK, D), lambda i: (i, 0))]
        + [pl.BlockSpec((BLK, D), lambda i, k=k: ((1 + k) * NB + i, 0))
           for k in range(S1)]
        + [pl.BlockSpec((BLK, D),
                        lambda i, m=m: ((1 + S1 + m) * NB + i, 0))
           for m in range(S1 * S2)]
        + [
            pl.BlockSpec((D, H1), lambda i: (0, 0)),
            pl.BlockSpec((H1, H2), lambda i: (0, 0)),
            pl.BlockSpec((1, H2, OUT), lambda i, t=t: (t, 0, 0)),
            prev_spec,
        ]
    )
    return pl.pallas_call(
        _tc_body_build(t),
        grid=(NB,),
        in_specs=in_specs,
        out_specs=pl.BlockSpec((BLK, OUT), lambda i: (i, 0)),
        out_shape=jax.ShapeDtypeStruct((B, OUT), jnp.float32),
    )


_tc_net_t = [_tc_net_build(t) for t in range(T)]


def kernel(x, nodes, nbr1, nbr2, W1, W2, Wp, bp):
    table = x.reshape(T * N, D)
    nodes_i = nodes.astype(jnp.int32)
    oidx = jnp.asarray(_OIDX)
    pad = jnp.asarray(_PADIDX)
    wp3 = Wp.reshape(T, H2, OUT)

    prev = bp.reshape(1, OUT)
    for t in range(T):
        idx_t = jnp.concatenate(
            [nodes_i, nbr1[t].astype(jnp.int32),
             nbr2[t].astype(jnp.int32), pad]) + (t * N)
        hg = _sc_gather(table, idx_t, oidx)
        args = [hg] * 16 + [W1, W2, wp3, prev]
        prev = _tc_net_t[t](*args)
    return prev
